# baseline (device time: 13166 ns/iter reference)
import jax
import jax.numpy as jnp
from jax import lax
from jax.experimental import pallas as pl
from jax.experimental.pallas import tpu as pltpu

N_DEV = 4

_SEND_ORDER = (2, 1, 3)
_RECV_ORDER = (1, 3, 2)


def kernel(x, w_mat):
    m_per, k = x.shape
    n = w_mat.shape[1]
    n_per = n // N_DEV

    def body(x_ref, w_hbm, out_ref, w_vmem, y_buf, copy_sems, send_sems,
             recv_sems):
        my = lax.axis_index("i")
        my_rows = pl.ds(my * m_per, m_per)

        copies = []
        for j in range(N_DEV):
            cp = pltpu.make_async_copy(
                w_hbm.at[:, j * n_per:(j + 1) * n_per],
                w_vmem.at[j],
                copy_sems.at[j],
            )
            cp.start()
            copies.append(cp)

        barrier_sem = pltpu.get_barrier_semaphore()
        for d in range(1, N_DEV):
            peer = lax.rem(my + d, N_DEV)
            pl.semaphore_signal(
                barrier_sem, inc=1,
                device_id=(peer,), device_id_type=pl.DeviceIdType.MESH,
            )
        pl.semaphore_wait(barrier_sem, N_DEV - 1)
        for cp in copies:
            cp.wait()

        x_val = x_ref[:, :]

        def chunk(dest):
            yc = jnp.dot(x_val, w_vmem[dest, :, :],
                         preferred_element_type=jnp.float32)
            return jnp.maximum(yc, 0.0).astype(jnp.bfloat16)

        rdmas = []
        for slot, d in enumerate(_SEND_ORDER):
            dest = lax.rem(my + d, N_DEV)
            y_buf[slot, :, :] = chunk(dest)
            rdma = pltpu.make_async_remote_copy(
                src_ref=y_buf.at[slot],
                dst_ref=out_ref.at[my_rows, :],
                send_sem=send_sems.at[slot],
                recv_sem=recv_sems.at[my],
                device_id=(dest,),
                device_id_type=pl.DeviceIdType.MESH,
            )
            rdma.start()
            rdmas.append(rdma)

        out_ref[my_rows, :] = chunk(my)

        for d in _RECV_ORDER:
            src = lax.rem(my - d + N_DEV, N_DEV)
            recv = pltpu.make_async_remote_copy(
                src_ref=y_buf.at[0],
                dst_ref=out_ref.at[pl.ds(src * m_per, m_per), :],
                send_sem=send_sems.at[0],
                recv_sem=recv_sems.at[src],
                device_id=(my,),
                device_id_type=pl.DeviceIdType.MESH,
            )
            recv.wait_recv()
        for rdma in rdmas:
            rdma.wait_send()

    return pl.pallas_call(
        body,
        out_shape=jax.ShapeDtypeStruct((N_DEV * m_per, n_per), jnp.bfloat16),
        in_specs=[
            pl.BlockSpec(memory_space=pltpu.VMEM),
            pl.BlockSpec(memory_space=pl.ANY),
        ],
        out_specs=pl.BlockSpec(memory_space=pltpu.VMEM),
        scratch_shapes=[
            pltpu.VMEM((N_DEV, k, n_per), jnp.float32),
            pltpu.VMEM((N_DEV - 1, m_per, n_per), jnp.bfloat16),
            pltpu.SemaphoreType.DMA((N_DEV,)),
            pltpu.SemaphoreType.DMA((N_DEV - 1,)),
            pltpu.SemaphoreType.DMA((N_DEV,)),
        ],
        compiler_params=pltpu.CompilerParams(collective_id=0),
    )(x, w_mat)


# device time: 12872 ns/iter; 1.0228x vs baseline; 1.0228x over previous
import jax
import jax.numpy as jnp
from jax import lax
from jax.experimental import pallas as pl
from jax.experimental.pallas import tpu as pltpu

N_DEV = 4

_SEND_ORDER = (2, 1, 3)
_RECV_ORDER = (1, 3, 2)


def kernel(x, w_mat):
    m_per, k = x.shape
    n = w_mat.shape[1]
    n_per = n // N_DEV

    def body(x_hbm, w_hbm, out_ref, x_vmem, w_vmem, y_buf, copy_sems,
             send_sems, recv_sems):
        my = lax.axis_index("i")
        my_rows = pl.ds(my * m_per, m_per)

        x_cp = pltpu.make_async_copy(x_hbm, x_vmem, copy_sems.at[0])
        w_cp = pltpu.make_async_copy(w_hbm, w_vmem, copy_sems.at[1])
        x_cp.start()
        w_cp.start()

        barrier_sem = pltpu.get_barrier_semaphore()
        for d in range(1, N_DEV):
            peer = lax.rem(my + d, N_DEV)
            pl.semaphore_signal(
                barrier_sem, inc=1,
                device_id=(peer,), device_id_type=pl.DeviceIdType.MESH,
            )
        pl.semaphore_wait(barrier_sem, N_DEV - 1)
        x_cp.wait()
        w_cp.wait()

        x_val = x_vmem[:, :]

        def chunk(dest):
            yc = jnp.dot(x_val, w_vmem[:, pl.ds(dest * n_per, n_per)],
                         preferred_element_type=jnp.float32)
            return jnp.maximum(yc, 0.0).astype(jnp.bfloat16)

        rdmas = []
        for slot, d in enumerate(_SEND_ORDER):
            dest = lax.rem(my + d, N_DEV)
            y_buf[slot, :, :] = chunk(dest)
            rdma = pltpu.make_async_remote_copy(
                src_ref=y_buf.at[slot],
                dst_ref=out_ref.at[my_rows, :],
                send_sem=send_sems.at[slot],
                recv_sem=recv_sems.at[my],
                device_id=(dest,),
                device_id_type=pl.DeviceIdType.MESH,
            )
            rdma.start()
            rdmas.append(rdma)

        out_ref[my_rows, :] = chunk(my)

        for d in _RECV_ORDER:
            src = lax.rem(my - d + N_DEV, N_DEV)
            recv = pltpu.make_async_remote_copy(
                src_ref=y_buf.at[0],
                dst_ref=out_ref.at[pl.ds(src * m_per, m_per), :],
                send_sem=send_sems.at[0],
                recv_sem=recv_sems.at[src],
                device_id=(my,),
                device_id_type=pl.DeviceIdType.MESH,
            )
            recv.wait_recv()
        for rdma in rdmas:
            rdma.wait_send()

    return pl.pallas_call(
        body,
        out_shape=jax.ShapeDtypeStruct((N_DEV * m_per, n_per), jnp.bfloat16),
        in_specs=[
            pl.BlockSpec(memory_space=pl.ANY),
            pl.BlockSpec(memory_space=pl.ANY),
        ],
        out_specs=pl.BlockSpec(memory_space=pltpu.VMEM),
        scratch_shapes=[
            pltpu.VMEM((m_per, k), jnp.float32),
            pltpu.VMEM((k, n), jnp.float32),
            pltpu.VMEM((N_DEV - 1, m_per, n_per), jnp.bfloat16),
            pltpu.SemaphoreType.DMA((2,)),
            pltpu.SemaphoreType.DMA((N_DEV - 1,)),
            pltpu.SemaphoreType.DMA((N_DEV,)),
        ],
        compiler_params=pltpu.CompilerParams(collective_id=0),
    )(x, w_mat)
